# speedup vs baseline: 1.6554x; 1.6554x over previous
import jax
import jax.numpy as jnp
from jax.experimental import pallas as pl

_B, _N, _D = 32, 20, 2048
_K = 4
_TO = 256


def _select_body(x_ref, neg_ref):
    xb = x_ref[0]
    g = jax.lax.dot_general(xb, xb, (((1,), (1,)), ((), ())),
                            preferred_element_type=jnp.float32)
    inner = -2.0 * g
    sq = jnp.sum(xb * xb, axis=1)
    adj = (sq[None, :] + inner) + sq[:, None]
    a_col = adj[:, :, None]
    a_row = adj[:, None, :]
    m_prime = jax.lax.broadcasted_iota(jnp.int32, (_N, _N, _N), 2)
    m_self = jax.lax.broadcasted_iota(jnp.int32, (_N, _N, _N), 1)
    better = (a_row < a_col) | ((a_row == a_col) & (m_prime < m_self))
    cnt = jnp.sum(better.astype(jnp.int32), axis=2)
    neg_ref[0] = jnp.where(cnt < _K, 0.0, 1e30).astype(jnp.float32)


def _main_body(x2_ref, w_ref, bias_ref, neg_ref, out_ref):
    x2 = x2_ref[...]
    w = w_ref[...]
    wa = w[:, :_D]
    wb = w[:, _D:]
    u = jax.lax.dot_general(x2, wa - wb, (((1,), (1,)), ((), ())),
                            preferred_element_type=jnp.float32)
    v = jax.lax.dot_general(x2, wb, (((1,), (1,)), ((), ())),
                            preferred_element_type=jnp.float32)
    bias = bias_ref[0]
    for b in range(_B):
        ub = u[b * _N:(b + 1) * _N]
        vb = v[b * _N:(b + 1) * _N]
        negb = neg_ref[b]
        t = vb[None, :, :] - negb[:, :, None]
        maxv = jnp.max(t, axis=1)
        r = jnp.max(ub + maxv, axis=0)
        out_ref[b, :] = jnp.maximum(r + bias, 0.0)


def kernel(x, W, b):
    x2 = x.reshape(_B * _N, _D)
    neg = pl.pallas_call(
        _select_body,
        grid=(_B,),
        in_specs=[pl.BlockSpec((1, _N, _D), lambda i: (i, 0, 0))],
        out_specs=pl.BlockSpec((1, _N, _N), lambda i: (i, 0, 0)),
        out_shape=jax.ShapeDtypeStruct((_B, _N, _N), jnp.float32),
    )(x)
    out = pl.pallas_call(
        _main_body,
        grid=(_D // _TO,),
        in_specs=[
            pl.BlockSpec((_B * _N, _D), lambda i: (0, 0)),
            pl.BlockSpec((_TO, 2 * _D), lambda i: (i, 0)),
            pl.BlockSpec((1, _TO), lambda i: (0, i)),
            pl.BlockSpec((_B, _N, _N), lambda i: (0, 0, 0)),
        ],
        out_specs=pl.BlockSpec((_B, _TO), lambda i: (0, i)),
        out_shape=jax.ShapeDtypeStruct((_B, _D), jnp.float32),
    )(x2, W, b.reshape(1, _D), neg)
    return out.reshape(_B, _D, 1, 1)


# speedup vs baseline: 2.0481x; 1.2372x over previous
import jax
import jax.numpy as jnp
from jax.experimental import pallas as pl

_B, _N, _D = 32, 20, 2048
_K = 4
_NP = 24
_TO = 256


def _select_body(x_ref, p_ref):
    xb = x_ref[0]
    g = jax.lax.dot_general(xb, xb, (((1,), (1,)), ((), ())),
                            preferred_element_type=jnp.float32)
    inner = -2.0 * g
    sq = jnp.sum(xb * xb, axis=1)
    adj = (sq[None, :] + inner) + sq[:, None]
    lane = jax.lax.broadcasted_iota(jnp.int32, (_N, _N), 1)
    cnt = jnp.zeros((_N, _N), jnp.int32)
    for mp in range(_N):
        col = adj[:, mp:mp + 1]
        hit = (col < adj) | ((col == adj) & (mp < lane))
        cnt = cnt + hit.astype(jnp.int32)
    p_ref[0] = jnp.zeros((_K * _NP, _N), jnp.float32)
    for j in range(_K):
        p_ref[0, _NP * j:_NP * j + _N, :] = (cnt == j).astype(jnp.float32)


def _main_body(x2_ref, w_ref, bias_ref, p_ref, out_ref):
    x2 = x2_ref[...]
    w = w_ref[...]
    wa = w[:, :_D]
    wb = w[:, _D:]
    u = jax.lax.dot_general(x2, wa - wb, (((1,), (1,)), ((), ())),
                            preferred_element_type=jnp.float32)
    v = jax.lax.dot_general(x2, wb, (((1,), (1,)), ((), ())),
                            preferred_element_type=jnp.float32)
    bias = bias_ref[0]
    for b in range(_B):
        ub = u[b * _N:(b + 1) * _N]
        vb = v[b * _N:(b + 1) * _N]
        pb = p_ref[b]
        gv = jax.lax.dot_general(pb, vb, (((1,), (0,)), ((), ())),
                                 preferred_element_type=jnp.float32)
        m01 = jnp.maximum(gv[0:_N], gv[_NP:_NP + _N])
        m23 = jnp.maximum(gv[2 * _NP:2 * _NP + _N], gv[3 * _NP:3 * _NP + _N])
        maxv = jnp.maximum(m01, m23)
        r = jnp.max(ub + maxv, axis=0)
        out_ref[b, :] = jnp.maximum(r + bias, 0.0)


def kernel(x, W, b):
    x2 = x.reshape(_B * _N, _D)
    p = pl.pallas_call(
        _select_body,
        grid=(_B,),
        in_specs=[pl.BlockSpec((1, _N, _D), lambda i: (i, 0, 0))],
        out_specs=pl.BlockSpec((1, _K * _NP, _N), lambda i: (i, 0, 0)),
        out_shape=jax.ShapeDtypeStruct((_B, _K * _NP, _N), jnp.float32),
    )(x)
    out = pl.pallas_call(
        _main_body,
        grid=(_D // _TO,),
        in_specs=[
            pl.BlockSpec((_B * _N, _D), lambda i: (0, 0)),
            pl.BlockSpec((_TO, 2 * _D), lambda i: (i, 0)),
            pl.BlockSpec((1, _TO), lambda i: (0, i)),
            pl.BlockSpec((_B, _K * _NP, _N), lambda i: (0, 0, 0)),
        ],
        out_specs=pl.BlockSpec((_B, _TO), lambda i: (0, i)),
        out_shape=jax.ShapeDtypeStruct((_B, _D), jnp.float32),
    )(x2, W, b.reshape(1, _D), p)
    return out.reshape(_B, _D, 1, 1)


# speedup vs baseline: 4.2496x; 2.0749x over previous
import jax
import jax.numpy as jnp
from jax.experimental import pallas as pl

_B, _N, _D = 32, 20, 2048
_K = 4
_NP = 24
_TO = 256


def _select_body(x_ref, p_ref):
    blocks = []
    for b in range(_B):
        xb = x_ref[b]
        g = jax.lax.dot_general(xb, xb, (((1,), (1,)), ((), ())),
                                preferred_element_type=jnp.float32)
        sq = jnp.sum(xb * xb, axis=1)
        blocks.append((sq[:, None] + (-2.0 * g)) + sq[None, :])
    adj = jnp.concatenate(blocks, axis=1)
    srow = jax.lax.broadcasted_iota(jnp.int32, (_N, _B * _N), 0)
    cnt = jnp.zeros((_N, _B * _N), jnp.int32)
    for mp in range(_N):
        row = adj[mp:mp + 1, :]
        hit = (row < adj) | ((row == adj) & (mp < srow))
        cnt = cnt + hit.astype(jnp.int32)
    for b in range(_B):
        cb = cnt[:, b * _N:(b + 1) * _N]
        for j in range(_K):
            p_ref[b, :, _NP * j:_NP * j + _N] = (cb == j).astype(jnp.float32)


def _main_body(x2_ref, w_ref, bias_ref, p_ref, out_ref):
    x2 = x2_ref[...]
    w = w_ref[...]
    wa = w[:, :_D]
    wb = w[:, _D:]
    u = jax.lax.dot_general(x2, wa - wb, (((1,), (1,)), ((), ())),
                            preferred_element_type=jnp.float32)
    v = jax.lax.dot_general(x2, wb, (((1,), (1,)), ((), ())),
                            preferred_element_type=jnp.float32)
    bias = bias_ref[0]
    for b in range(_B):
        ub = u[b * _N:(b + 1) * _N]
        vb = v[b * _N:(b + 1) * _N]
        pt = p_ref[b]
        gv = jax.lax.dot_general(pt, vb, (((0,), (0,)), ((), ())),
                                 preferred_element_type=jnp.float32)
        m01 = jnp.maximum(gv[0:_N], gv[_NP:_NP + _N])
        m23 = jnp.maximum(gv[2 * _NP:2 * _NP + _N], gv[3 * _NP:3 * _NP + _N])
        maxv = jnp.maximum(m01, m23)
        r = jnp.max(ub + maxv, axis=0)
        out_ref[b, :] = jnp.maximum(r + bias, 0.0)


def kernel(x, W, b):
    x2 = x.reshape(_B * _N, _D)
    p = pl.pallas_call(
        _select_body,
        grid=(1,),
        in_specs=[pl.BlockSpec((_B, _N, _D), lambda i: (0, 0, 0))],
        out_specs=pl.BlockSpec((_B, _N, _K * _NP), lambda i: (0, 0, 0)),
        out_shape=jax.ShapeDtypeStruct((_B, _N, _K * _NP), jnp.float32),
    )(x)
    out = pl.pallas_call(
        _main_body,
        grid=(_D // _TO,),
        in_specs=[
            pl.BlockSpec((_B * _N, _D), lambda i: (0, 0)),
            pl.BlockSpec((_TO, 2 * _D), lambda i: (i, 0)),
            pl.BlockSpec((1, _TO), lambda i: (0, i)),
            pl.BlockSpec((_B, _N, _K * _NP), lambda i: (0, 0, 0)),
        ],
        out_specs=pl.BlockSpec((_B, _TO), lambda i: (0, i)),
        out_shape=jax.ShapeDtypeStruct((_B, _D), jnp.float32),
    )(x2, W, b.reshape(1, _D), p)
    return out.reshape(_B, _D, 1, 1)


# speedup vs baseline: 4.3863x; 1.0322x over previous
import jax
import jax.numpy as jnp
from jax.experimental import pallas as pl
from jax.experimental.pallas import tpu as pltpu

_B, _N, _D = 32, 20, 2048
_K = 4
_NP = 24
_TO = 256


def _select(x2, p_ref):
    blocks = []
    sqs = []
    for b in range(_B):
        xb = x2[b * _N:(b + 1) * _N]
        g = jax.lax.dot_general(xb, xb, (((1,), (1,)), ((), ())),
                                preferred_element_type=jnp.float32)
        sq = jnp.sum(xb * xb, axis=1)
        blocks.append((sq[:, None] + (-2.0 * g)) + sq[None, :])
    adj = jnp.concatenate(blocks, axis=1)
    srow = jax.lax.broadcasted_iota(jnp.int32, (_N, _B * _N), 0)
    cnt = jnp.zeros((_N, _B * _N), jnp.int32)
    for mp in range(_N):
        row = adj[mp:mp + 1, :]
        hit = (row < adj) | ((row == adj) & (mp < srow))
        cnt = cnt + hit.astype(jnp.int32)
    for b in range(_B):
        cb = cnt[:, b * _N:(b + 1) * _N]
        for j in range(_K):
            p_ref[b, :, _NP * j:_NP * j + _N] = (cb == j).astype(jnp.float32)


def _body(x2_ref, w_ref, bias_ref, out_ref, p_ref):
    x2 = x2_ref[...]

    @pl.when(pl.program_id(0) == 0)
    def _():
        _select(x2, p_ref)

    w = w_ref[...]
    wa = w[:, :_D]
    wb = w[:, _D:]
    u = jax.lax.dot_general(x2, wa - wb, (((1,), (1,)), ((), ())),
                            preferred_element_type=jnp.float32)
    v = jax.lax.dot_general(x2, wb, (((1,), (1,)), ((), ())),
                            preferred_element_type=jnp.float32)
    bias = bias_ref[0]
    for b in range(_B):
        ub = u[b * _N:(b + 1) * _N]
        vb = v[b * _N:(b + 1) * _N]
        pt = p_ref[b]
        gv = jax.lax.dot_general(pt, vb, (((0,), (0,)), ((), ())),
                                 preferred_element_type=jnp.float32)
        m01 = jnp.maximum(gv[0:_N], gv[_NP:_NP + _N])
        m23 = jnp.maximum(gv[2 * _NP:2 * _NP + _N], gv[3 * _NP:3 * _NP + _N])
        maxv = jnp.maximum(m01, m23)
        r = jnp.max(ub + maxv, axis=0)
        out_ref[b, :] = jnp.maximum(r + bias, 0.0)


def kernel(x, W, b):
    x2 = x.reshape(_B * _N, _D)
    out = pl.pallas_call(
        _body,
        grid=(_D // _TO,),
        in_specs=[
            pl.BlockSpec((_B * _N, _D), lambda i: (0, 0)),
            pl.BlockSpec((_TO, 2 * _D), lambda i: (i, 0)),
            pl.BlockSpec((1, _TO), lambda i: (0, i)),
        ],
        out_specs=pl.BlockSpec((_B, _TO), lambda i: (0, i)),
        out_shape=jax.ShapeDtypeStruct((_B, _D), jnp.float32),
        scratch_shapes=[pltpu.VMEM((_B, _N, _K * _NP), jnp.float32)],
    )(x2, W, b.reshape(1, _D))
    return out.reshape(_B, _D, 1, 1)


# speedup vs baseline: 4.6078x; 1.0505x over previous
import jax
import jax.numpy as jnp
from jax.experimental import pallas as pl
from jax.experimental.pallas import tpu as pltpu

_B, _N, _D = 32, 20, 2048
_K = 4
_NP = 24
_TO = 256


def _select(x2, p_ref):
    blocks = []
    for b in range(_B):
        xb = x2[b * _N:(b + 1) * _N]
        g = jax.lax.dot_general(xb, xb, (((1,), (1,)), ((), ())),
                                preferred_element_type=jnp.float32)
        sq = jnp.sum(xb * xb, axis=1)
        blocks.append((sq[:, None] + (-2.0 * g)) + sq[None, :])
    adj = jnp.concatenate(blocks, axis=1)
    srow = jax.lax.broadcasted_iota(jnp.int32, (_N, _B * _N), 0)
    cnt = jnp.zeros((_N, _B * _N), jnp.int32)
    for mp in range(_N):
        row = adj[mp:mp + 1, :]
        hit = (row < adj) | ((row == adj) & (mp < srow))
        cnt = cnt + hit.astype(jnp.int32)
    for b in range(_B):
        cb = cnt[:, b * _N:(b + 1) * _N]
        for j in range(_K):
            p_ref[b, :, _NP * j:_NP * j + _N] = (cb == j).astype(jnp.float32)


def _body(x2_hbm, w_ref, bias_ref, out_ref, x2_vmem, p_ref, sem):
    @pl.when(pl.program_id(0) == 0)
    def _():
        copy = pltpu.make_async_copy(x2_hbm, x2_vmem, sem)
        copy.start()
        copy.wait()
        _select(x2_vmem[...], p_ref)

    x2 = x2_vmem[...]
    w = w_ref[...]
    wa = w[:, :_D]
    wb = w[:, _D:]
    u = jax.lax.dot_general(x2, wa - wb, (((1,), (1,)), ((), ())),
                            preferred_element_type=jnp.float32)
    v = jax.lax.dot_general(x2, wb, (((1,), (1,)), ((), ())),
                            preferred_element_type=jnp.float32)
    bias = bias_ref[0]
    for b in range(_B):
        ub = u[b * _N:(b + 1) * _N]
        vb = v[b * _N:(b + 1) * _N]
        pt = p_ref[b]
        gv = jax.lax.dot_general(pt, vb, (((0,), (0,)), ((), ())),
                                 preferred_element_type=jnp.float32)
        m01 = jnp.maximum(gv[0:_N], gv[_NP:_NP + _N])
        m23 = jnp.maximum(gv[2 * _NP:2 * _NP + _N], gv[3 * _NP:3 * _NP + _N])
        maxv = jnp.maximum(m01, m23)
        r = jnp.max(ub + maxv, axis=0)
        out_ref[b, :] = jnp.maximum(r + bias, 0.0)


def kernel(x, W, b):
    x2 = x.reshape(_B * _N, _D)
    out = pl.pallas_call(
        _body,
        grid=(_D // _TO,),
        in_specs=[
            pl.BlockSpec(memory_space=pl.ANY),
            pl.BlockSpec((_TO, 2 * _D), lambda i: (i, 0)),
            pl.BlockSpec((1, _TO), lambda i: (0, i)),
        ],
        out_specs=pl.BlockSpec((_B, _TO), lambda i: (0, i)),
        out_shape=jax.ShapeDtypeStruct((_B, _D), jnp.float32),
        scratch_shapes=[
            pltpu.VMEM((_B * _N, _D), jnp.float32),
            pltpu.VMEM((_B, _N, _K * _NP), jnp.float32),
            pltpu.SemaphoreType.DMA,
        ],
    )(x2, W, b.reshape(1, _D))
    return out.reshape(_B, _D, 1, 1)
